# Initial kernel scaffold; baseline (speedup 1.0000x reference)
import jax
import jax.numpy as jnp
from jax.experimental import pallas as pl


def kernel(x, A, B):
    raise NotImplementedError("write your pallas kernel here")


# speedup vs baseline: 1.3281x; 1.3281x over previous
import jax
import jax.numpy as jnp
from jax.experimental import pallas as pl
from jax.experimental.pallas import tpu as pltpu

_MiB = 1024 * 1024


def _lora_body(x_ref, a_ref, b_ref, o_ref):
    xa = jnp.dot(x_ref[...], a_ref[...], preferred_element_type=jnp.float32)
    o_ref[...] = jnp.dot(xa, b_ref[...], preferred_element_type=jnp.float32)


def kernel(x, A, B, alpha=16.0):
    M, K = x.shape
    R, N = B.shape
    assert A.shape == (K, R)
    out_dtype = x.dtype

    B_s = jnp.asarray(alpha, jnp.float32) * B.astype(jnp.float32)

    tm = min(512, M)
    grid = (pl.cdiv(M, tm),)

    flops = 2 * M * K * R + 2 * M * R * N
    bytes_accessed = (M * K + K * R + R * N + M * N) * 4
    cost = pl.CostEstimate(flops=flops, transcendentals=0,
                           bytes_accessed=bytes_accessed)

    out = pl.pallas_call(
        _lora_body,
        out_shape=jax.ShapeDtypeStruct((M, N), out_dtype),
        grid=grid,
        in_specs=[
            pl.BlockSpec((tm, K), lambda i: (i, 0)),
            pl.BlockSpec((K, R), lambda i: (0, 0)),
            pl.BlockSpec((R, N), lambda i: (0, 0)),
        ],
        out_specs=pl.BlockSpec((tm, N), lambda i: (i, 0)),
        compiler_params=pltpu.CompilerParams(
            dimension_semantics=("parallel",),
            vmem_limit_bytes=48 * _MiB,
        ),
        cost_estimate=cost,
    )(x, A.astype(jnp.float32), B_s)
    return out
